# baseline (speedup 1.0000x reference)
import functools

import jax
import jax.numpy as jnp
from jax import lax
from jax.experimental import pallas as pl
from jax.experimental.pallas import tpu as pltpu
from jax.experimental.pallas import tpu_sc as plsc

B, L, D, K = 256, 128, 128, 32
NC, NS = 2, 16
NW = NC * NS
BPW = B // NW
CH = D // 16


def _sc_agg_body(feats_hbm, head_hbm, out_hbm,
                 feats_v, acc_v, head_v, cnt_v, ones_v, zcnt_v, rdeg_v, crd_v,
                 acc_sh, cnt_sh):
    cid = lax.axis_index("c")
    sid = lax.axis_index("s")
    wid = sid * NC + cid

    iota16 = lax.iota(jnp.int32, 16)

    def init_const(t, _):
        ones_v[t, :] = jnp.ones((16,), jnp.float32)
        zcnt_v[t, :] = jnp.zeros((16,), jnp.float32)
        return 0
    lax.fori_loop(0, L, init_const, 0)

    def batch_body(q, _):
        bb = wid * BPW + q
        pltpu.sync_copy(feats_hbm.at[bb], feats_v)
        pltpu.sync_copy(head_hbm.at[bb], head_v)
        pltpu.sync_copy(feats_v, acc_sh.at[sid])
        pltpu.sync_copy(zcnt_v, cnt_sh.at[sid])

        pltpu.sync_copy(feats_v, acc_sh.at[sid].at[head_v], add=True)
        pltpu.sync_copy(ones_v, cnt_sh.at[sid].at[head_v], add=True)
        pltpu.sync_copy(acc_sh.at[sid], acc_v)
        pltpu.sync_copy(cnt_sh.at[sid], cnt_v)

        def coef_body(t, _):
            h16 = head_v[pl.ds(t * 16, 16)]
            i16 = iota16 + t * 16
            hh = plsc.load_gather(head_v, [h16])
            p = (h16 != i16).astype(jnp.int32)
            m = jnp.where(hh == i16, p, 0)
            coef = (2 * p - m - 1).astype(jnp.float32)
            cnt16 = plsc.load_gather(cnt_v, [i16, jnp.zeros((16,), jnp.int32)])
            rdeg = 1.0 / (cnt16 + coef + 1.0)
            rdeg_v[pl.ds(t * 16, 16)] = rdeg
            crd_v[pl.ds(t * 16, 16)] = coef * rdeg
            return 0
        lax.fori_loop(0, L // 16, coef_body, 0)

        def row_body(i, _):
            isplat = jnp.full((16,), i, jnp.int32)
            rb = plsc.load_gather(rdeg_v, [isplat])
            cb = plsc.load_gather(crd_v, [isplat])
            hb = plsc.load_gather(head_v, [isplat])
            for j in range(CH):
                src = acc_v[i, pl.ds(j * 16, 16)]
                par = plsc.load_gather(feats_v, [hb, iota16 + j * 16])
                acc_v[i, pl.ds(j * 16, 16)] = src * rb + par * cb
            return 0
        lax.fori_loop(0, L, row_body, 0)

        pltpu.sync_copy(acc_v, out_hbm.at[bb])
        return 0

    lax.fori_loop(0, BPW, batch_body, 0)


def _sc_agg(feats, head):
    mesh = plsc.VectorSubcoreMesh(core_axis_name="c", subcore_axis_name="s")
    return pl.kernel(
        _sc_agg_body,
        out_type=jax.ShapeDtypeStruct((B, L, D), jnp.float32),
        mesh=mesh,
        compiler_params=pltpu.CompilerParams(needs_layout_passes=False),
        scratch_types=[
            pltpu.VMEM((L, D), jnp.float32),
            pltpu.VMEM((L, D), jnp.float32),
            pltpu.VMEM((L,), jnp.int32),
            pltpu.VMEM((L, 16), jnp.float32),
            pltpu.VMEM((L, 16), jnp.float32),
            pltpu.VMEM((L, 16), jnp.float32),
            pltpu.VMEM((L,), jnp.float32),
            pltpu.VMEM((L,), jnp.float32),
            pltpu.VMEM_SHARED((NS, L, D), jnp.float32),
            pltpu.VMEM_SHARED((NS, L, 16), jnp.float32),
        ],
    )(feats, head)


GB = 8


def _tc_body(x_ref, w_ref, b_ref, c_ref, o_ref):
    x = x_ref[...].reshape(GB * L, D)
    h = jnp.dot(x, w_ref[...], preferred_element_type=jnp.float32) + b_ref[...]
    h = jnp.maximum(h, 0.0)
    s = lax.dot_general(h, c_ref[...], (((1,), (1,)), ((), ())),
                        preferred_element_type=jnp.float32)
    mx = jnp.max(s, axis=-1, keepdims=True)
    e = jnp.exp(s - mx)
    o_ref[...] = (e / jnp.sum(e, axis=-1, keepdims=True)).reshape(GB, L, K)


def _tc_tail(aggn, W_gnn, b_gnn, centroids):
    return pl.pallas_call(
        _tc_body,
        grid=(B // GB,),
        in_specs=[
            pl.BlockSpec((GB, L, D), lambda i: (i, 0, 0)),
            pl.BlockSpec((D, D), lambda i: (0, 0)),
            pl.BlockSpec((1, D), lambda i: (0, 0)),
            pl.BlockSpec((K, D), lambda i: (0, 0)),
        ],
        out_specs=pl.BlockSpec((GB, L, K), lambda i: (i, 0, 0)),
        out_shape=jax.ShapeDtypeStruct((B, L, K), jnp.float32),
    )(aggn, W_gnn, b_gnn.reshape(1, D), centroids)


def kernel(feats, tokens, aspect, pos, post, head, deprel, sen_len, adk,
           pos_mask, word_mask, aspect_pos_start, aspect_pos_end,
           plain_text, text_list, W_gnn, b_gnn, centroids):
    aggn = _sc_agg(feats, head.astype(jnp.int32))
    return _tc_tail(aggn, W_gnn, b_gnn, centroids)


# speedup vs baseline: 1.3523x; 1.3523x over previous
import jax
import jax.numpy as jnp
from jax import lax
from jax.experimental import pallas as pl
from jax.experimental.pallas import tpu as pltpu
from jax.experimental.pallas import tpu_sc as plsc

B, L, D, K = 256, 128, 128, 32
NC, NS = 2, 16
NW = NC * NS
BPW = B // NW
CH = D // 16


def _sc_agg_body(feats2_hbm, head_hbm, out_hbm,
                 fv0, fv1, fv2, hv0, hv1, hv2, pv0, pv1, cv0, cv1,
                 gi0, gi1, ones_v, zcnt_v, rdeg_v, crd_v,
                 acc_sh, cnt_sh, *sems):
    cid = lax.axis_index("c")
    sid = lax.axis_index("s")
    wid = sid * NC + cid
    base = wid * BPW

    fvs, hvs = [fv0, fv1, fv2], [hv0, hv1, hv2]
    pvs, cvs, gis = [pv0, pv1], [cv0, cv1], [gi0, gi1]
    (sA0, sA1, sA2, sH0, sH1, sH2, sB0, sB1, sC0, sC1,
     sD0, sD1, sP0, sP1, sF0, sF1, sF2) = sems
    sAs, sHs, sFs = [sA0, sA1, sA2], [sH0, sH1, sH2], [sF0, sF1, sF2]
    sPs = [sP0, sP1]

    iota16 = lax.iota(jnp.int32, 16)

    def init_const(t, _):
        ones_v[t, :] = jnp.ones((16,), jnp.float32)
        zcnt_v[t, :] = jnp.zeros((16,), jnp.float32)
        return 0
    lax.fori_loop(0, L, init_const, 0)

    def issue_load(q):
        r = q % 3
        a = pltpu.async_copy(feats2_hbm.at[pl.ds((base + q) * L, L)],
                             fvs[r], sAs[r])
        h = pltpu.async_copy(head_hbm.at[base + q], hvs[r], sHs[r])
        return a, h

    def issue_par(q):
        r, p = q % 3, q % 2
        off = (base + q) * L
        for t in range(L // 16):
            gis[p][pl.ds(t * 16, 16)] = hvs[r][pl.ds(t * 16, 16)] + off
        return pltpu.async_copy(feats2_hbm.at[gis[p]], pvs[p], sPs[p])

    def issue_init(q):
        r = q % 3
        b1 = pltpu.async_copy(fvs[r], acc_sh.at[sid], sB0)
        b2 = pltpu.async_copy(zcnt_v, cnt_sh.at[sid], sB1)
        return b1, b2

    def issue_scatter(q):
        r = q % 3
        c1 = pltpu.async_copy(fvs[r], acc_sh.at[sid].at[hvs[r]], sC0,
                              add=True)
        c2 = pltpu.async_copy(ones_v, cnt_sh.at[sid].at[hvs[r]], sC1,
                              add=True)
        return c1, c2

    def issue_readback(q):
        r, p = q % 3, q % 2
        d1 = pltpu.async_copy(acc_sh.at[sid], fvs[r], sD0)
        d2 = pltpu.async_copy(cnt_sh.at[sid], cvs[p], sD1)
        return d1, d2

    def coef_phase(q):
        r, p = q % 3, q % 2
        for t in range(L // 16):
            h16 = hvs[r][pl.ds(t * 16, 16)]
            i16 = iota16 + t * 16
            hh = plsc.load_gather(hvs[r], [h16])
            pf = (h16 != i16).astype(jnp.int32)
            m = jnp.where(hh == i16, pf, 0)
            coef = (2 * pf - m - 1).astype(jnp.float32)
            cnt16 = plsc.load_gather(cvs[p], [i16, jnp.zeros((16,), jnp.int32)])
            rdeg = 1.0 / (cnt16 + coef + 1.0)
            rdeg_v[pl.ds(t * 16, 16)] = rdeg
            crd_v[pl.ds(t * 16, 16)] = coef * rdeg

    def finalize_rows(q, lo, hi):
        r, p = q % 3, q % 2
        av, pv = fvs[r], pvs[p]

        def row2(k, _):
            i = k * 2 + lo
            for u in range(2):
                isplat = jnp.full((16,), i + u, jnp.int32)
                rb = plsc.load_gather(rdeg_v, [isplat])
                cb = plsc.load_gather(crd_v, [isplat])
                for j in range(CH):
                    a = av[i + u, pl.ds(j * 16, 16)]
                    pr = pv[i + u, pl.ds(j * 16, 16)]
                    av[i + u, pl.ds(j * 16, 16)] = a * rb + pr * cb
            return 0
        lax.fori_loop(0, (hi - lo) // 2, row2, 0)

    ad = [None] * BPW
    hd = [None] * BPW
    fd = [None] * BPW
    pd = [None] * BPW
    ad[0], hd[0] = issue_load(0)
    ad[1], hd[1] = issue_load(1)
    ad[0].wait(); hd[0].wait()
    b1, b2 = issue_init(0)
    pd[0] = issue_par(0)
    b1.wait(); b2.wait()
    c1, c2 = issue_scatter(0)
    c1.wait(); c2.wait()
    d1, d2 = issue_readback(0)
    d1.wait(); d2.wait()

    for q in range(BPW):
        nxt = q + 1 < BPW
        if nxt:
            ad[q + 1].wait(); hd[q + 1].wait()
            b1, b2 = issue_init(q + 1)
            pd[q + 1] = issue_par(q + 1)
        coef_phase(q)
        if nxt:
            b1.wait(); b2.wait()
            c1, c2 = issue_scatter(q + 1)
        if q + 2 < BPW:
            if q >= 1:
                fd[q - 1].wait()
            ad[q + 2], hd[q + 2] = issue_load(q + 2)
        pd[q].wait()
        finalize_rows(q, 0, L // 2)
        if nxt:
            c1.wait(); c2.wait()
            d1, d2 = issue_readback(q + 1)
        finalize_rows(q, L // 2, L)
        fd[q] = pltpu.async_copy(fvs[q % 3],
                                 out_hbm.at[pl.ds((base + q) * L, L)],
                                 sFs[q % 3])
        if nxt:
            d1.wait(); d2.wait()

    fd[BPW - 2].wait()
    fd[BPW - 1].wait()


def _sc_agg(feats2, head):
    mesh = plsc.VectorSubcoreMesh(core_axis_name="c", subcore_axis_name="s")
    return pl.kernel(
        _sc_agg_body,
        out_type=jax.ShapeDtypeStruct((B * L, D), jnp.float32),
        mesh=mesh,
        compiler_params=pltpu.CompilerParams(needs_layout_passes=False,
                                             use_tc_tiling_on_sc=False),
        scratch_types=[
            pltpu.VMEM((L, D), jnp.float32),
            pltpu.VMEM((L, D), jnp.float32),
            pltpu.VMEM((L, D), jnp.float32),
            pltpu.VMEM((L,), jnp.int32),
            pltpu.VMEM((L,), jnp.int32),
            pltpu.VMEM((L,), jnp.int32),
            pltpu.VMEM((L, D), jnp.float32),
            pltpu.VMEM((L, D), jnp.float32),
            pltpu.VMEM((L, 16), jnp.float32),
            pltpu.VMEM((L, 16), jnp.float32),
            pltpu.VMEM((L,), jnp.int32),
            pltpu.VMEM((L,), jnp.int32),
            pltpu.VMEM((L, 16), jnp.float32),
            pltpu.VMEM((L, 16), jnp.float32),
            pltpu.VMEM((L,), jnp.float32),
            pltpu.VMEM((L,), jnp.float32),
            pltpu.VMEM_SHARED((NS, L, D), jnp.float32),
            pltpu.VMEM_SHARED((NS, L, 16), jnp.float32),
        ] + [pltpu.SemaphoreType.DMA] * 17,
    )(feats2, head)


GB = 8


def _tc_body(x_ref, w_ref, b_ref, c_ref, o_ref):
    x = x_ref[...].reshape(GB * L, D)
    h = jnp.dot(x, w_ref[...], preferred_element_type=jnp.float32) + b_ref[...]
    h = jnp.maximum(h, 0.0)
    s = lax.dot_general(h, c_ref[...], (((1,), (1,)), ((), ())),
                        preferred_element_type=jnp.float32)
    mx = jnp.max(s, axis=-1, keepdims=True)
    e = jnp.exp(s - mx)
    o_ref[...] = (e / jnp.sum(e, axis=-1, keepdims=True)).reshape(GB, L, K)


def _tc_tail(aggn, W_gnn, b_gnn, centroids):
    return pl.pallas_call(
        _tc_body,
        grid=(B // GB,),
        in_specs=[
            pl.BlockSpec((GB, L, D), lambda i: (i, 0, 0)),
            pl.BlockSpec((D, D), lambda i: (0, 0)),
            pl.BlockSpec((1, D), lambda i: (0, 0)),
            pl.BlockSpec((K, D), lambda i: (0, 0)),
        ],
        out_specs=pl.BlockSpec((GB, L, K), lambda i: (i, 0, 0)),
        out_shape=jax.ShapeDtypeStruct((B, L, K), jnp.float32),
    )(aggn, W_gnn, b_gnn.reshape(1, D), centroids)


def kernel(feats, tokens, aspect, pos, post, head, deprel, sen_len, adk,
           pos_mask, word_mask, aspect_pos_start, aspect_pos_end,
           plain_text, text_list, W_gnn, b_gnn, centroids):
    h32 = head.astype(jnp.int32)
    aggn = _sc_agg(feats.reshape(B * L, D), h32).reshape(B, L, D)
    return _tc_tail(aggn, W_gnn, b_gnn, centroids)
